# baseline (speedup 1.0000x reference)
import functools

import jax
import jax.numpy as jnp
from jax import lax
from jax.experimental import pallas as pl
from jax.experimental.pallas import tpu as pltpu
from jax.experimental.pallas import tpu_sc as plsc

N_NODES = 10000
N_EDGES = 320000
D = 128
LANES = 16
NC, NS = 2, 16
NW = NC * NS
EDGE_BATCH = 80
NBUF = 4
N_BATCH = 128
PER_TILE = N_BATCH * EDGE_BATCH
EDGES_PAD = NW * PER_TILE
N_PAD = 10240
ROWS_PER_TILE = N_PAD // NS

_sc_mesh = plsc.VectorSubcoreMesh(core_axis_name="c", subcore_axis_name="s")


@functools.partial(
    pl.kernel,
    mesh=_sc_mesh,
    out_type=jax.ShapeDtypeStruct((NC * N_PAD, D), jnp.float32),
    scratch_types=(
        [pltpu.VMEM((2, EDGE_BATCH), jnp.int32) for _ in range(NBUF)]
        + [pltpu.VMEM((EDGE_BATCH,), jnp.float32) for _ in range(NBUF)]
        + [pltpu.VMEM((EDGE_BATCH, D), jnp.float32) for _ in range(NBUF)]
        + [pltpu.VMEM_SHARED((N_PAD, D), jnp.float32)]
        + [pltpu.SemaphoreType.DMA for _ in range(2 * NBUF)]
    ),
)
def _sc_scatter(x_hbm, packed_hbm, w_hbm, zero_hbm, out_hbm, *refs):
    idx = refs[0:NBUF]
    wbuf = refs[NBUF:2 * NBUF]
    rows = refs[2 * NBUF:3 * NBUF]
    acc = refs[3 * NBUF]
    isem = refs[3 * NBUF + 1:3 * NBUF + 1 + NBUF]
    gsem = refs[3 * NBUF + 1 + NBUF:3 * NBUF + 1 + 2 * NBUF]
    cid = lax.axis_index("c")
    sid = lax.axis_index("s")
    wid = cid * NS + sid

    row0 = sid * ROWS_PER_TILE
    pltpu.sync_copy(zero_hbm.at[pl.ds(row0, ROWS_PER_TILE)],
                    acc.at[pl.ds(row0, ROWS_PER_TILE)])
    plsc.subcore_barrier()

    nb0 = wid * N_BATCH

    def start_idx(i, b):
        pltpu.make_async_copy(packed_hbm.at[nb0 + i], idx[b], isem[b]).start()
        pltpu.make_async_copy(w_hbm.at[pl.ds((nb0 + i) * EDGE_BATCH, EDGE_BATCH)],
                              wbuf[b], isem[b]).start()

    def wait_idx(b):
        pltpu.make_async_copy(packed_hbm.at[nb0], idx[b], isem[b]).wait()
        pltpu.make_async_copy(w_hbm.at[pl.ds(0, EDGE_BATCH)],
                              wbuf[b], isem[b]).wait()

    def start_gather(b):
        pltpu.make_async_copy(x_hbm.at[idx[b].at[0]], rows[b], gsem[b]).start()

    def wait_gather(b):
        pltpu.make_async_copy(x_hbm.at[idx[b].at[0]], rows[b], gsem[b]).wait()

    def scale(b):
        rv = rows[b]

        def group_body(g, c2):
            wv = wbuf[b][pl.ds(g * LANES, LANES)]
            for k in range(LANES):
                wb = jnp.full((LANES,), wv[k], dtype=jnp.float32)
                e = g * LANES + k
                for j in range(D // LANES):
                    sl = pl.ds(j * LANES, LANES)
                    rv[e, sl] = rv[e, sl] * wb
            return c2

        lax.fori_loop(0, EDGE_BATCH // LANES, group_body, 0)

    for j in range(NBUF):
        start_idx(j, j)
    for j in range(NBUF - 1):
        wait_idx(j)
        start_gather(j)

    def ring_body(t, carry):
        for b in range(NBUF):
            i = NBUF * t + b
            prv = (b - 1) % NBUF
            wait_gather(b)

            @pl.when(i + NBUF - 1 < N_BATCH)
            def _():
                wait_idx(prv)
                start_gather(prv)

            scale(b)
            pltpu.sync_copy(rows[b], acc.at[idx[b].at[1]], add=True)

            @pl.when(i + NBUF < N_BATCH)
            def _():
                start_idx(i + NBUF, b)
        return carry

    lax.fori_loop(0, N_BATCH // NBUF, ring_body, 0)

    plsc.subcore_barrier()
    pltpu.sync_copy(acc.at[pl.ds(row0, ROWS_PER_TILE)],
                    out_hbm.at[pl.ds(cid * N_PAD + row0, ROWS_PER_TILE)])


def _tc_body(p0_ref, p1_ref, w_ref, b_ref, o_ref):
    s = p0_ref[...] + p1_ref[...]
    o_ref[...] = lax.dot_general(
        s, w_ref[...], (((1,), (1,)), ((), ())),
        preferred_element_type=jnp.float32) + b_ref[...]


BLOCK_N = 1000

_tc_combine = pl.pallas_call(
    _tc_body,
    grid=(N_NODES // BLOCK_N,),
    in_specs=[
        pl.BlockSpec((BLOCK_N, D), lambda i: (i, 0)),
        pl.BlockSpec((BLOCK_N, D), lambda i: (i, 0)),
        pl.BlockSpec((D, D), lambda i: (0, 0)),
        pl.BlockSpec((1, D), lambda i: (0, 0)),
    ],
    out_specs=pl.BlockSpec((BLOCK_N, D), lambda i: (i, 0)),
    out_shape=jax.ShapeDtypeStruct((N_NODES, D), jnp.float32),
)


def kernel(x, edge_index, edge_weight, W, b):
    src = edge_index[0].astype(jnp.int32)
    dst = edge_index[1].astype(jnp.int32)
    pad = EDGES_PAD - N_EDGES
    src = jnp.concatenate([src, jnp.zeros((pad,), jnp.int32)])
    dst = jnp.concatenate([dst, jnp.zeros((pad,), jnp.int32)])
    w = jnp.concatenate([edge_weight.astype(jnp.float32),
                         jnp.zeros((pad,), jnp.float32)])
    nbt = EDGES_PAD // EDGE_BATCH
    packed = jnp.stack([src.reshape(nbt, EDGE_BATCH),
                        dst.reshape(nbt, EDGE_BATCH)], axis=1)
    zeros = jnp.zeros((N_PAD, D), jnp.float32)
    partials = _sc_scatter(x, packed, w, zeros)
    return _tc_combine(partials[:N_NODES], partials[N_PAD:N_PAD + N_NODES],
                       W, b.reshape(1, D))


# speedup vs baseline: 1.0017x; 1.0017x over previous
import functools

import jax
import jax.numpy as jnp
from jax import lax
from jax.experimental import pallas as pl
from jax.experimental.pallas import tpu as pltpu
from jax.experimental.pallas import tpu_sc as plsc

N_NODES = 10000
N_EDGES = 320000
D = 128
LANES = 16
NC, NS = 2, 16
NW = NC * NS
EDGE_BATCH = 80
HALF = EDGE_BATCH // 2
NBUF = 4
N_BATCH = 128
PER_TILE = N_BATCH * EDGE_BATCH
EDGES_PAD = NW * PER_TILE
N_PAD = 10240
ROWS_PER_TILE = N_PAD // NS

_sc_mesh = plsc.VectorSubcoreMesh(core_axis_name="c", subcore_axis_name="s")


@functools.partial(
    pl.kernel,
    mesh=_sc_mesh,
    out_type=jax.ShapeDtypeStruct((NC * N_PAD, D), jnp.float32),
    scratch_types=(
        [pltpu.VMEM((2, EDGE_BATCH), jnp.int32) for _ in range(NBUF)]
        + [pltpu.VMEM((EDGE_BATCH,), jnp.float32) for _ in range(NBUF)]
        + [pltpu.VMEM((EDGE_BATCH, D), jnp.float32) for _ in range(NBUF)]
        + [pltpu.VMEM_SHARED((N_PAD, D), jnp.float32)]
        + [pltpu.SemaphoreType.DMA for _ in range(2 * NBUF)]
    ),
)
def _sc_scatter(x_hbm, packed_hbm, w_hbm, zero_hbm, out_hbm, *refs):
    idx = refs[0:NBUF]
    wbuf = refs[NBUF:2 * NBUF]
    rows = refs[2 * NBUF:3 * NBUF]
    acc = refs[3 * NBUF]
    isem = refs[3 * NBUF + 1:3 * NBUF + 1 + NBUF]
    gsem = refs[3 * NBUF + 1 + NBUF:3 * NBUF + 1 + 2 * NBUF]
    cid = lax.axis_index("c")
    sid = lax.axis_index("s")
    wid = cid * NS + sid

    row0 = sid * ROWS_PER_TILE
    pltpu.sync_copy(zero_hbm.at[pl.ds(row0, ROWS_PER_TILE)],
                    acc.at[pl.ds(row0, ROWS_PER_TILE)])
    plsc.subcore_barrier()

    nb0 = wid * N_BATCH

    def start_idx(i, b):
        pltpu.make_async_copy(packed_hbm.at[nb0 + i], idx[b], isem[b]).start()
        pltpu.make_async_copy(w_hbm.at[pl.ds((nb0 + i) * EDGE_BATCH, EDGE_BATCH)],
                              wbuf[b], isem[b]).start()

    def wait_idx(b):
        pltpu.make_async_copy(packed_hbm.at[nb0], idx[b], isem[b]).wait()
        pltpu.make_async_copy(w_hbm.at[pl.ds(0, EDGE_BATCH)],
                              wbuf[b], isem[b]).wait()

    def start_gather(b):
        pltpu.make_async_copy(x_hbm.at[idx[b].at[0, pl.ds(0, HALF)]],
                              rows[b].at[pl.ds(0, HALF)], gsem[b]).start()
        pltpu.make_async_copy(x_hbm.at[idx[b].at[0, pl.ds(HALF, HALF)]],
                              rows[b].at[pl.ds(HALF, HALF)], gsem[b]).start()

    def wait_gather(b):
        pltpu.make_async_copy(x_hbm.at[idx[b].at[0, pl.ds(0, HALF)]],
                              rows[b].at[pl.ds(0, HALF)], gsem[b]).wait()
        pltpu.make_async_copy(x_hbm.at[idx[b].at[0, pl.ds(HALF, HALF)]],
                              rows[b].at[pl.ds(HALF, HALF)], gsem[b]).wait()

    def scale(b):
        rv = rows[b]

        def group_body(g, c2):
            wv = wbuf[b][pl.ds(g * LANES, LANES)]
            for k in range(LANES):
                wb = jnp.full((LANES,), wv[k], dtype=jnp.float32)
                e = g * LANES + k
                for j in range(D // LANES):
                    sl = pl.ds(j * LANES, LANES)
                    rv[e, sl] = rv[e, sl] * wb
            return c2

        lax.fori_loop(0, EDGE_BATCH // LANES, group_body, 0)

    for j in range(NBUF):
        start_idx(j, j)
    for j in range(NBUF - 1):
        wait_idx(j)
        start_gather(j)

    def ring_body(t, carry):
        for b in range(NBUF):
            i = NBUF * t + b
            prv = (b - 1) % NBUF
            wait_gather(b)

            @pl.when(i + NBUF - 1 < N_BATCH)
            def _():
                wait_idx(prv)
                start_gather(prv)

            scale(b)
            pltpu.sync_copy(rows[b], acc.at[idx[b].at[1]], add=True)

            @pl.when(i + NBUF < N_BATCH)
            def _():
                start_idx(i + NBUF, b)
        return carry

    lax.fori_loop(0, N_BATCH // NBUF, ring_body, 0)

    plsc.subcore_barrier()
    pltpu.sync_copy(acc.at[pl.ds(row0, ROWS_PER_TILE)],
                    out_hbm.at[pl.ds(cid * N_PAD + row0, ROWS_PER_TILE)])


def _tc_body(p0_ref, p1_ref, w_ref, b_ref, o_ref):
    s = p0_ref[...] + p1_ref[...]
    o_ref[...] = lax.dot_general(
        s, w_ref[...], (((1,), (1,)), ((), ())),
        preferred_element_type=jnp.float32) + b_ref[...]


BLOCK_N = 1000

_tc_combine = pl.pallas_call(
    _tc_body,
    grid=(N_NODES // BLOCK_N,),
    in_specs=[
        pl.BlockSpec((BLOCK_N, D), lambda i: (i, 0)),
        pl.BlockSpec((BLOCK_N, D), lambda i: (i, 0)),
        pl.BlockSpec((D, D), lambda i: (0, 0)),
        pl.BlockSpec((1, D), lambda i: (0, 0)),
    ],
    out_specs=pl.BlockSpec((BLOCK_N, D), lambda i: (i, 0)),
    out_shape=jax.ShapeDtypeStruct((N_NODES, D), jnp.float32),
)


def kernel(x, edge_index, edge_weight, W, b):
    src = edge_index[0].astype(jnp.int32)
    dst = edge_index[1].astype(jnp.int32)
    pad = EDGES_PAD - N_EDGES
    src = jnp.concatenate([src, jnp.zeros((pad,), jnp.int32)])
    dst = jnp.concatenate([dst, jnp.zeros((pad,), jnp.int32)])
    w = jnp.concatenate([edge_weight.astype(jnp.float32),
                         jnp.zeros((pad,), jnp.float32)])
    nbt = EDGES_PAD // EDGE_BATCH
    packed = jnp.stack([src.reshape(nbt, EDGE_BATCH),
                        dst.reshape(nbt, EDGE_BATCH)], axis=1)
    zeros = jnp.zeros((N_PAD, D), jnp.float32)
    partials = _sc_scatter(x, packed, w, zeros)
    return _tc_combine(partials[:N_NODES], partials[N_PAD:N_PAD + N_NODES],
                       W, b.reshape(1, D))
